# baseline (speedup 1.0000x reference)
import functools

import jax
import jax.numpy as jnp
from jax import lax
from jax.experimental import pallas as pl
from jax.experimental.pallas import tpu as pltpu
from jax.experimental.pallas import tpu_sc as plsc


def _dot(a, b):
    return jnp.dot(a.astype(jnp.bfloat16), b.astype(jnp.bfloat16),
                   preferred_element_type=jnp.float32)


def _enc_body(x_ref, pd_ref, w1_ref, b1_ref, w2_ref, b2_ref, cb_ref,
              ze_ref, idx_ref):
    x2d = x_ref[0]
    lq = x2d.shape[1] // 4
    xcat = _dot(x2d, pd_ref[...])
    x0 = xcat[:, 0:lq]
    x1 = xcat[:, lq:2 * lq]
    x2 = xcat[:, 2 * lq:3 * lq]
    x3 = xcat[:, 3 * lq:4 * lq]
    c_in = x0.shape[0]
    zc = jnp.zeros((c_in, 1), jnp.float32)
    x3m = jnp.concatenate([zc, x3[:, :-1]], axis=1)
    x0p = jnp.concatenate([x0[:, 1:], zc], axis=1)
    xe = jnp.concatenate([x3m, x0, x1, x2], axis=0)
    xo = jnp.concatenate([x1, x2, x3, x0p], axis=0)
    w1 = w1_ref[...]
    b1 = b1_ref[...]
    z1e = jnp.maximum(_dot(w1, xe) + b1, 0.0)
    z1o = jnp.maximum(_dot(w1, xo) + b1, 0.0)
    h = z1e.shape[0]
    zc2 = jnp.zeros((h, 1), jnp.float32)
    z1om = jnp.concatenate([zc2, z1o[:, :-1]], axis=1)
    z1ep = jnp.concatenate([z1e[:, 1:], zc2], axis=1)
    z2cat = jnp.concatenate([z1om, z1e, z1o, z1ep], axis=0)
    ze = _dot(w2_ref[...], z2cat) + b2_ref[...]
    ze_ref[0] = ze
    cb = cb_ref[...]
    sc = _dot(cb, ze)
    x2s = jnp.sum(ze * ze, axis=0, keepdims=True)
    y2 = jnp.sum(cb * cb, axis=1, keepdims=True)
    d = jnp.sqrt(jnp.maximum(x2s + y2 - 2.0 * sc, 0.0))
    dmin = jnp.min(d, axis=0, keepdims=True)
    k = cb.shape[0]
    ii = lax.broadcasted_iota(jnp.int32, (k, d.shape[1]), 0)
    idx = jnp.min(jnp.where(d == dmin, ii, jnp.int32(2**30)), axis=0,
                  keepdims=True)
    idx_ref[0] = idx


def _dec_body(idx_ref, cb_ref, ae_ref, ao_ref, b13_ref, b02_ref,
              db1_ref, db2_ref, pe_ref, out_ref):
    idx = idx_ref[0]
    cb = cb_ref[...]
    kk, dd = cb.shape
    lq = idx.shape[1]
    hf = lq // 2
    r5 = lax.broadcasted_iota(jnp.int32, (lq, hf), 0)
    c5 = lax.broadcasted_iota(jnp.int32, (lq, hf), 1)
    s_e = (r5 == 2 * c5).astype(jnp.float32)
    s_o = (r5 == 2 * c5 + 1).astype(jnp.float32)
    idx_f = idx.astype(jnp.float32)
    hi = lax.Precision.HIGHEST
    idx_e = jnp.dot(idx_f, s_e, precision=hi).astype(jnp.int32)
    idx_o = jnp.dot(idx_f, s_o, precision=hi).astype(jnp.int32)
    ii0 = lax.broadcasted_iota(jnp.int32, (kk, hf), 0)
    m_et = (ii0 == idx_e).astype(jnp.bfloat16)
    m_ot = (ii0 == idx_o).astype(jnp.bfloat16)
    g_e = _dot(jnp.transpose(m_et), cb)
    g_o = jnp.transpose(m_ot)
    g_o = _dot(g_o, cb)
    z = jnp.concatenate([g_e, g_o], axis=1)
    d_ = z.shape[0]
    zc = jnp.zeros((d_, 1), jnp.float32)
    z_m1 = jnp.concatenate([zc, z[:, :-1]], axis=1)
    z_p1 = jnp.concatenate([z[:, 1:], zc], axis=1)
    db1 = db1_ref[...]
    he = jnp.maximum(_dot(ae_ref[...], jnp.concatenate([z, z_m1], axis=0)) + db1, 0.0)
    ho = jnp.maximum(_dot(ao_ref[...], jnp.concatenate([z_p1, z], axis=0)) + db1, 0.0)
    hdim = he.shape[0]
    hc = jnp.zeros((hdim, 1), jnp.float32)
    ho_m1 = jnp.concatenate([hc, ho[:, :-1]], axis=1)
    he_p1 = jnp.concatenate([he[:, 1:], hc], axis=1)
    db2 = db2_ref[...]
    b13 = b13_ref[...]
    b02 = b02_ref[...]
    xr0 = _dot(b13, jnp.concatenate([he, ho_m1], axis=0)) + db2
    xr1 = _dot(b02, jnp.concatenate([ho, he], axis=0)) + db2
    xr2 = _dot(b13, jnp.concatenate([ho, he], axis=0)) + db2
    xr3 = _dot(b02, jnp.concatenate([he_p1, ho], axis=0)) + db2
    xcat = jnp.concatenate([xr0, xr1, xr2, xr3], axis=1)
    out_ref[0] = _dot(xcat, pe_ref[...])


def _wprep_body(w1r_ref, p1_ref, w2r_ref, p2_ref, a1r_ref, q13_ref, q02_ref,
                b2r_ref, r13_ref, r02_ref,
                w1o_ref, w2o_ref, ae_ref, ao_ref, b13_ref, b02_ref):
    bf = jnp.bfloat16
    w1o_ref[...] = _dot(w1r_ref[...], p1_ref[...]).astype(bf)
    w2o_ref[...] = _dot(w2r_ref[...], p2_ref[...]).astype(bf)
    h = ae_ref.shape[0]
    t13 = _dot(a1r_ref[...], q13_ref[...])
    ae_ref[...] = jnp.concatenate(
        [jnp.transpose(t13[:, 0:h]), jnp.transpose(t13[:, h:2 * h])],
        axis=1).astype(bf)
    t02 = _dot(a1r_ref[...], q02_ref[...])
    ao_ref[...] = jnp.concatenate(
        [jnp.transpose(t02[:, 0:h]), jnp.transpose(t02[:, h:2 * h])],
        axis=1).astype(bf)
    c = b13_ref.shape[0]
    u13 = _dot(b2r_ref[...], r13_ref[...])
    b13_ref[...] = jnp.concatenate(
        [jnp.transpose(u13[:, 0:c]), jnp.transpose(u13[:, c:2 * c])],
        axis=1).astype(bf)
    u02 = _dot(b2r_ref[...], r02_ref[...])
    b02_ref[...] = jnp.concatenate(
        [jnp.transpose(u02[:, 0:c]), jnp.transpose(u02[:, c:2 * c])],
        axis=1).astype(bf)


def _tap_perm(n, g):
    ui = lax.broadcasted_iota(jnp.int32, (n, n), 0)
    qi = lax.broadcasted_iota(jnp.int32, (n, n), 1)
    return (qi == (ui % 4) * g + ui // 4).astype(jnp.bfloat16)


def _tap_sel(n, m, t_lo, t_hi):
    qi = lax.broadcasted_iota(jnp.int32, (n, 2 * m), 0)
    pi = lax.broadcasted_iota(jnp.int32, (n, 2 * m), 1)
    return jnp.where(pi < m, qi == pi * 4 + t_lo,
                     qi == (pi - m) * 4 + t_hi).astype(jnp.bfloat16)


def _sc_gather(table, idx):
    n, d = idx.shape[0], table.shape[1]
    nc, ns = 2, 16
    nw = nc * ns
    bpw = n // nw
    mesh = plsc.VectorSubcoreMesh(core_axis_name="c", subcore_axis_name="s")

    @functools.partial(
        pl.kernel,
        mesh=mesh,
        out_type=jax.ShapeDtypeStruct((n, d), jnp.float32),
        scratch_types=[
            pltpu.VMEM((bpw,), jnp.int32),
            pltpu.VMEM((bpw, d), jnp.float32),
            pltpu.SemaphoreType.DMA,
        ],
    )
    def k(table_hbm, idx_hbm, out_hbm, idx_v, rows_v, sem):
        wid = lax.axis_index("s") * nc + lax.axis_index("c")
        base = wid * bpw
        pltpu.sync_copy(idx_hbm.at[pl.ds(base, bpw)], idx_v)
        pltpu.async_copy(table_hbm.at[idx_v], rows_v, sem).wait()
        pltpu.sync_copy(rows_v, out_hbm.at[pl.ds(base, bpw)])

    return k(table, idx)


def kernel(x, conv1_w, conv1_b, conv2_w, conv2_b, codebook,
           deconv1_w, deconv1_b, deconv2_w, deconv2_b):
    B, C_in, L = x.shape
    H = conv1_w.shape[0]
    D, K = conv2_w.shape[0], codebook.shape[0]
    Lq = L // 4
    f32 = jnp.float32

    ui = lax.broadcasted_iota(jnp.int32, (L, L), 0)
    qi = lax.broadcasted_iota(jnp.int32, (L, L), 1)
    pd = (qi == (ui % 4) * Lq + ui // 4).astype(jnp.bfloat16)
    pe = (ui == (qi % 4) * Lq + qi // 4).astype(jnp.bfloat16)
    b1c = conv1_b.reshape(H, 1)
    b2c = conv2_b.reshape(D, 1)

    bf = jnp.bfloat16
    full = lambda *s: pl.BlockSpec(s, lambda: tuple(0 for _ in s))
    w1cat, w2cat, ae, ao, b13, b02 = pl.pallas_call(
        _wprep_body,
        grid=(),
        in_specs=[
            full(H, 4 * C_in), full(4 * C_in, 4 * C_in),
            full(D, 4 * H), full(4 * H, 4 * H),
            full(D, 4 * H), full(4 * H, 2 * H), full(4 * H, 2 * H),
            full(H, 4 * C_in), full(4 * C_in, 2 * C_in), full(4 * C_in, 2 * C_in),
        ],
        out_specs=[
            full(H, 4 * C_in), full(D, 4 * H), full(H, 2 * D), full(H, 2 * D),
            full(C_in, 2 * H), full(C_in, 2 * H),
        ],
        out_shape=[
            jax.ShapeDtypeStruct((H, 4 * C_in), bf),
            jax.ShapeDtypeStruct((D, 4 * H), bf),
            jax.ShapeDtypeStruct((H, 2 * D), bf),
            jax.ShapeDtypeStruct((H, 2 * D), bf),
            jax.ShapeDtypeStruct((C_in, 2 * H), bf),
            jax.ShapeDtypeStruct((C_in, 2 * H), bf),
        ],
    )(conv1_w.reshape(H, 4 * C_in), _tap_perm(4 * C_in, C_in),
      conv2_w.reshape(D, 4 * H), _tap_perm(4 * H, H),
      deconv1_w.reshape(D, 4 * H), _tap_sel(4 * H, H, 1, 3),
      _tap_sel(4 * H, H, 0, 2),
      deconv2_w.reshape(H, 4 * C_in), _tap_sel(4 * C_in, C_in, 1, 3),
      _tap_sel(4 * C_in, C_in, 0, 2))

    ze, idx3 = pl.pallas_call(
        _enc_body,
        grid=(B,),
        in_specs=[
            pl.BlockSpec((1, C_in, L), lambda b: (b, 0, 0)),
            pl.BlockSpec((L, L), lambda b: (0, 0)),
            pl.BlockSpec((H, 4 * C_in), lambda b: (0, 0)),
            pl.BlockSpec((H, 1), lambda b: (0, 0)),
            pl.BlockSpec((D, 4 * H), lambda b: (0, 0)),
            pl.BlockSpec((D, 1), lambda b: (0, 0)),
            pl.BlockSpec((K, D), lambda b: (0, 0)),
        ],
        out_specs=[
            pl.BlockSpec((1, D, Lq), lambda b: (b, 0, 0)),
            pl.BlockSpec((1, 1, Lq), lambda b: (b, 0, 0)),
        ],
        out_shape=[
            jax.ShapeDtypeStruct((B, D, Lq), f32),
            jax.ShapeDtypeStruct((B, 1, Lq), jnp.int32),
        ],
    )(x, pd, w1cat, b1c, w2cat, b2c, codebook)

    encoding_indices = idx3.reshape(B * Lq)

    zq_flat = _sc_gather(codebook, encoding_indices)
    z_q = zq_flat.reshape(B, D, Lq)

    db1 = deconv1_b.reshape(H, 1)
    db2 = deconv2_b.reshape(C_in, 1)

    x_recon = pl.pallas_call(
        _dec_body,
        grid=(B,),
        in_specs=[
            pl.BlockSpec((1, 1, Lq), lambda b: (b, 0, 0)),
            pl.BlockSpec((K, D), lambda b: (0, 0)),
            pl.BlockSpec((H, 2 * D), lambda b: (0, 0)),
            pl.BlockSpec((H, 2 * D), lambda b: (0, 0)),
            pl.BlockSpec((C_in, 2 * H), lambda b: (0, 0)),
            pl.BlockSpec((C_in, 2 * H), lambda b: (0, 0)),
            pl.BlockSpec((H, 1), lambda b: (0, 0)),
            pl.BlockSpec((C_in, 1), lambda b: (0, 0)),
            pl.BlockSpec((L, L), lambda b: (0, 0)),
        ],
        out_specs=pl.BlockSpec((1, C_in, L), lambda b: (b, 0, 0)),
        out_shape=jax.ShapeDtypeStruct((B, C_in, L), f32),
    )(idx3, codebook, ae, ao, b13, b02, db1, db2, pe)

    return (x_recon, z_q, encoding_indices, ze)


# speedup vs baseline: 1.0950x; 1.0950x over previous
import functools

import jax
import jax.numpy as jnp
from jax import lax
from jax.experimental import pallas as pl
from jax.experimental.pallas import tpu as pltpu
from jax.experimental.pallas import tpu_sc as plsc


def _dot(a, b):
    return jnp.dot(a.astype(jnp.bfloat16), b.astype(jnp.bfloat16),
                   preferred_element_type=jnp.float32)


def _enc_body(x_ref, pd_ref, w1_ref, b1_ref, w2_ref, b2_ref, cb_ref,
              ze_ref, idx_ref):
    x2d = x_ref[0]
    lq = x2d.shape[1] // 4
    xcat = _dot(x2d, pd_ref[...])
    x0 = xcat[:, 0:lq]
    x1 = xcat[:, lq:2 * lq]
    x2 = xcat[:, 2 * lq:3 * lq]
    x3 = xcat[:, 3 * lq:4 * lq]
    c_in = x0.shape[0]
    zc = jnp.zeros((c_in, 1), jnp.float32)
    x3m = jnp.concatenate([zc, x3[:, :-1]], axis=1)
    x0p = jnp.concatenate([x0[:, 1:], zc], axis=1)
    xe = jnp.concatenate([x3m, x0, x1, x2], axis=0)
    xo = jnp.concatenate([x1, x2, x3, x0p], axis=0)
    w1 = w1_ref[...]
    b1 = b1_ref[...]
    z1e = jnp.maximum(_dot(w1, xe) + b1, 0.0)
    z1o = jnp.maximum(_dot(w1, xo) + b1, 0.0)
    h = z1e.shape[0]
    zc2 = jnp.zeros((h, 1), jnp.float32)
    z1om = jnp.concatenate([zc2, z1o[:, :-1]], axis=1)
    z1ep = jnp.concatenate([z1e[:, 1:], zc2], axis=1)
    z2cat = jnp.concatenate([z1om, z1e, z1o, z1ep], axis=0)
    ze = _dot(w2_ref[...], z2cat) + b2_ref[...]
    ze_ref[0] = ze
    cb = cb_ref[...]
    sc = _dot(cb, ze)
    x2s = jnp.sum(ze * ze, axis=0, keepdims=True)
    y2 = jnp.sum(cb * cb, axis=1, keepdims=True)
    d = jnp.sqrt(jnp.maximum(x2s + y2 - 2.0 * sc, 0.0))
    dmin = jnp.min(d, axis=0, keepdims=True)
    k = cb.shape[0]
    ii = lax.broadcasted_iota(jnp.int32, (k, d.shape[1]), 0)
    idx = jnp.min(jnp.where(d == dmin, ii, jnp.int32(2**30)), axis=0,
                  keepdims=True)
    idx_ref[0] = idx


def _dec_body(idx_ref, cb_ref, ae_ref, ao_ref, b13_ref, b02_ref,
              db1_ref, db2_ref, pe_ref, out_ref):
    idx = idx_ref[0]
    cb = cb_ref[...]
    kk, dd = cb.shape
    lq = idx.shape[1]
    hf = lq // 2
    r5 = lax.broadcasted_iota(jnp.int32, (lq, hf), 0)
    c5 = lax.broadcasted_iota(jnp.int32, (lq, hf), 1)
    s_e = (r5 == 2 * c5).astype(jnp.float32)
    s_o = (r5 == 2 * c5 + 1).astype(jnp.float32)
    idx_f = idx.astype(jnp.float32)
    hi = lax.Precision.HIGHEST
    idx_e = jnp.dot(idx_f, s_e, precision=hi).astype(jnp.int32)
    idx_o = jnp.dot(idx_f, s_o, precision=hi).astype(jnp.int32)
    ii0 = lax.broadcasted_iota(jnp.int32, (kk, hf), 0)
    m_et = (ii0 == idx_e).astype(jnp.bfloat16)
    m_ot = (ii0 == idx_o).astype(jnp.bfloat16)
    g_e = _dot(jnp.transpose(m_et), cb)
    g_o = jnp.transpose(m_ot)
    g_o = _dot(g_o, cb)
    z = jnp.concatenate([g_e, g_o], axis=1)
    d_ = z.shape[0]
    zc = jnp.zeros((d_, 1), jnp.float32)
    z_m1 = jnp.concatenate([zc, z[:, :-1]], axis=1)
    z_p1 = jnp.concatenate([z[:, 1:], zc], axis=1)
    db1 = db1_ref[...]
    he = jnp.maximum(_dot(ae_ref[...], jnp.concatenate([z, z_m1], axis=0)) + db1, 0.0)
    ho = jnp.maximum(_dot(ao_ref[...], jnp.concatenate([z_p1, z], axis=0)) + db1, 0.0)
    hdim = he.shape[0]
    hc = jnp.zeros((hdim, 1), jnp.float32)
    ho_m1 = jnp.concatenate([hc, ho[:, :-1]], axis=1)
    he_p1 = jnp.concatenate([he[:, 1:], hc], axis=1)
    db2 = db2_ref[...]
    b13 = b13_ref[...]
    b02 = b02_ref[...]
    xr0 = _dot(b13, jnp.concatenate([he, ho_m1], axis=0)) + db2
    xr1 = _dot(b02, jnp.concatenate([ho, he], axis=0)) + db2
    xr2 = _dot(b13, jnp.concatenate([ho, he], axis=0)) + db2
    xr3 = _dot(b02, jnp.concatenate([he_p1, ho], axis=0)) + db2
    xcat = jnp.concatenate([xr0, xr1, xr2, xr3], axis=1)
    out_ref[0] = _dot(xcat, pe_ref[...])


def _sc_gather(table, idx):
    n, d = idx.shape[0], table.shape[1]
    nc, ns = 2, 16
    nw = nc * ns
    bpw = n // nw
    mesh = plsc.VectorSubcoreMesh(core_axis_name="c", subcore_axis_name="s")

    @functools.partial(
        pl.kernel,
        mesh=mesh,
        out_type=jax.ShapeDtypeStruct((n, d), jnp.float32),
        scratch_types=[
            pltpu.VMEM((bpw,), jnp.int32),
            pltpu.VMEM((bpw, d), jnp.float32),
            pltpu.SemaphoreType.DMA,
        ],
    )
    def k(table_hbm, idx_hbm, out_hbm, idx_v, rows_v, sem):
        wid = lax.axis_index("s") * nc + lax.axis_index("c")
        base = wid * bpw
        pltpu.sync_copy(idx_hbm.at[pl.ds(base, bpw)], idx_v)
        pltpu.async_copy(table_hbm.at[idx_v], rows_v, sem).wait()
        pltpu.sync_copy(rows_v, out_hbm.at[pl.ds(base, bpw)])

    return k(table, idx)


def kernel(x, conv1_w, conv1_b, conv2_w, conv2_b, codebook,
           deconv1_w, deconv1_b, deconv2_w, deconv2_b):
    B, C_in, L = x.shape
    H = conv1_w.shape[0]
    D, K = conv2_w.shape[0], codebook.shape[0]
    Lq = L // 4
    f32 = jnp.float32

    ui = lax.broadcasted_iota(jnp.int32, (L, L), 0)
    qi = lax.broadcasted_iota(jnp.int32, (L, L), 1)
    pd = (qi == (ui % 4) * Lq + ui // 4).astype(jnp.bfloat16)
    pe = (ui == (qi % 4) * Lq + qi // 4).astype(jnp.bfloat16)
    w1cat = conv1_w.transpose(0, 2, 1).reshape(H, 4 * C_in)
    w2cat = conv2_w.transpose(0, 2, 1).reshape(D, 4 * H)
    b1c = conv1_b.reshape(H, 1)
    b2c = conv2_b.reshape(D, 1)

    ze, idx3 = pl.pallas_call(
        _enc_body,
        grid=(B,),
        in_specs=[
            pl.BlockSpec((1, C_in, L), lambda b: (b, 0, 0)),
            pl.BlockSpec((L, L), lambda b: (0, 0)),
            pl.BlockSpec((H, 4 * C_in), lambda b: (0, 0)),
            pl.BlockSpec((H, 1), lambda b: (0, 0)),
            pl.BlockSpec((D, 4 * H), lambda b: (0, 0)),
            pl.BlockSpec((D, 1), lambda b: (0, 0)),
            pl.BlockSpec((K, D), lambda b: (0, 0)),
        ],
        out_specs=[
            pl.BlockSpec((1, D, Lq), lambda b: (b, 0, 0)),
            pl.BlockSpec((1, 1, Lq), lambda b: (b, 0, 0)),
        ],
        out_shape=[
            jax.ShapeDtypeStruct((B, D, Lq), f32),
            jax.ShapeDtypeStruct((B, 1, Lq), jnp.int32),
        ],
    )(x, pd, w1cat, b1c, w2cat, b2c, codebook)

    encoding_indices = idx3.reshape(B * Lq)

    zq_flat = _sc_gather(codebook, encoding_indices)
    z_q = zq_flat.reshape(B, D, Lq)

    at = deconv1_w.transpose(2, 1, 0)
    ae = jnp.concatenate([at[1], at[3]], axis=1)
    ao = jnp.concatenate([at[0], at[2]], axis=1)
    bt = deconv2_w.transpose(2, 1, 0)
    b13 = jnp.concatenate([bt[1], bt[3]], axis=1)
    b02 = jnp.concatenate([bt[0], bt[2]], axis=1)
    db1 = deconv1_b.reshape(H, 1)
    db2 = deconv2_b.reshape(C_in, 1)

    x_recon = pl.pallas_call(
        _dec_body,
        grid=(B,),
        in_specs=[
            pl.BlockSpec((1, 1, Lq), lambda b: (b, 0, 0)),
            pl.BlockSpec((K, D), lambda b: (0, 0)),
            pl.BlockSpec((H, 2 * D), lambda b: (0, 0)),
            pl.BlockSpec((H, 2 * D), lambda b: (0, 0)),
            pl.BlockSpec((C_in, 2 * H), lambda b: (0, 0)),
            pl.BlockSpec((C_in, 2 * H), lambda b: (0, 0)),
            pl.BlockSpec((H, 1), lambda b: (0, 0)),
            pl.BlockSpec((C_in, 1), lambda b: (0, 0)),
            pl.BlockSpec((L, L), lambda b: (0, 0)),
        ],
        out_specs=pl.BlockSpec((1, C_in, L), lambda b: (b, 0, 0)),
        out_shape=jax.ShapeDtypeStruct((B, C_in, L), f32),
    )(idx3, codebook, ae, ao, b13, b02, db1, db2, pe)

    return (x_recon, z_q, encoding_indices, ze)
